# Initial kernel scaffold; baseline (speedup 1.0000x reference)
import jax
import jax.numpy as jnp
from jax.experimental import pallas as pl


def kernel(edge_index, params, W1, b1, W2, b2, pW1, pb1, pW2, pb2, fW1, fb1, fW2, fb2):
    raise NotImplementedError("write your pallas kernel here")


# speedup vs baseline: 24.7413x; 24.7413x over previous
import functools

import jax
import jax.numpy as jnp
from jax import lax
from jax.experimental import pallas as pl
from jax.experimental.pallas import tpu as pltpu
from jax.experimental.pallas import tpu_sc as plsc

_N = 50000
_E = 1600000
_NC = 2
_NS = 16
_NW = _NC * _NS
_EW = _E // _NW
_B = 80
_K2 = 25
_K32 = 5
_NP = 50048
_RT = _NP // _NS


def _mesh():
    return plsc.VectorSubcoreMesh(core_axis_name="c", subcore_axis_name="s",
                                  num_cores=_NC, num_subcores=_NS)


_SC_PARAMS = pltpu.CompilerParams(use_tc_tiling_on_sc=False)


def _drain(descs):
    for d in descs:
        d.wait()


def _deg_body(k, e_ref, ones_ref, z_ref, oin_ref, oout_ref, *scratch):
    didx = scratch[:k]
    sidx = scratch[k:2 * k]
    ones, acc_i, acc_o, sem = scratch[2 * k:]
    steps = _EW // (_B * k)
    c = lax.axis_index("c")
    s = lax.axis_index("s")
    wid = c * _NS + s
    pltpu.sync_copy(z_ref.at[pl.ds(s * _RT, _RT)], acc_i.at[pl.ds(s * _RT, _RT)])
    pltpu.sync_copy(z_ref.at[pl.ds(s * _RT, _RT)], acc_o.at[pl.ds(s * _RT, _RT)])
    pltpu.sync_copy(ones_ref, ones)
    plsc.subcore_barrier()

    def step(g, carry):
        _drain([pltpu.async_copy(e_ref.at[1, wid, g, j], didx[j], sem)
                for j in range(k)]
               + [pltpu.async_copy(e_ref.at[0, wid, g, j], sidx[j], sem)
                  for j in range(k)])
        _drain([pltpu.async_copy(ones, acc_i.at[didx[j]], sem, add=True)
                for j in range(k)]
               + [pltpu.async_copy(ones, acc_o.at[sidx[j]], sem, add=True)
                  for j in range(k)])
        return carry

    lax.fori_loop(0, steps, step, 0)
    plsc.subcore_barrier()
    pltpu.sync_copy(acc_i.at[pl.ds(s * _RT, _RT)],
                    oin_ref.at[c, pl.ds(s * _RT, _RT)])
    pltpu.sync_copy(acc_o.at[pl.ds(s * _RT, _RT)],
                    oout_ref.at[c, pl.ds(s * _RT, _RT)])


def _make_deg_kernel(k=_K2):
    return pl.kernel(
        functools.partial(_deg_body, k),
        out_type=(jax.ShapeDtypeStruct((_NC, _NP), jnp.float32),
                  jax.ShapeDtypeStruct((_NC, _NP), jnp.float32)),
        mesh=_mesh(),
        compiler_params=_SC_PARAMS,
        scratch_types=(
            [pltpu.VMEM((_B,), jnp.int32) for _ in range(2 * k)]
            + [pltpu.VMEM((_B,), jnp.float32),
               pltpu.VMEM_SHARED((_NP,), jnp.float32),
               pltpu.VMEM_SHARED((_NP,), jnp.float32),
               pltpu.SemaphoreType.DMA]
        ),
    )


def _agg1_body(k, e_ref, tin_ref, tout_ref, z_ref, oin_ref, oout_ref,
               *scratch):
    didx = scratch[:k]
    sidx = scratch[k:2 * k]
    rin, rout, acc_i, acc_o, sem = scratch[2 * k:]
    steps = _EW // (_B * k)
    c = lax.axis_index("c")
    s = lax.axis_index("s")
    wid = c * _NS + s
    pltpu.sync_copy(z_ref.at[pl.ds(s * _RT, _RT)], acc_i.at[pl.ds(s * _RT, _RT)])
    pltpu.sync_copy(z_ref.at[pl.ds(s * _RT, _RT)], acc_o.at[pl.ds(s * _RT, _RT)])
    plsc.subcore_barrier()

    def step(g, carry):
        _drain([pltpu.async_copy(e_ref.at[0, wid, g, j], sidx[j], sem)
                for j in range(k)]
               + [pltpu.async_copy(e_ref.at[1, wid, g, j], didx[j], sem)
                  for j in range(k)])
        _drain([pltpu.async_copy(tin_ref.at[sidx[j]], rin.at[j], sem)
                for j in range(k)]
               + [pltpu.async_copy(tout_ref.at[sidx[j]], rout.at[j], sem)
                  for j in range(k)])
        _drain([pltpu.async_copy(rin.at[j], acc_i.at[didx[j]], sem, add=True)
                for j in range(k)]
               + [pltpu.async_copy(rout.at[j], acc_o.at[didx[j]], sem,
                                   add=True)
                  for j in range(k)])
        return carry

    lax.fori_loop(0, steps, step, 0)
    plsc.subcore_barrier()
    pltpu.sync_copy(acc_i.at[pl.ds(s * _RT, _RT)],
                    oin_ref.at[c, pl.ds(s * _RT, _RT)])
    pltpu.sync_copy(acc_o.at[pl.ds(s * _RT, _RT)],
                    oout_ref.at[c, pl.ds(s * _RT, _RT)])


def _make_agg1_kernel(k=_K2):
    return pl.kernel(
        functools.partial(_agg1_body, k),
        out_type=(jax.ShapeDtypeStruct((_NC, _NP), jnp.float32),
                  jax.ShapeDtypeStruct((_NC, _NP), jnp.float32)),
        mesh=_mesh(),
        compiler_params=_SC_PARAMS,
        scratch_types=(
            [pltpu.VMEM((_B,), jnp.int32) for _ in range(2 * k)]
            + [pltpu.VMEM((k, _B), jnp.float32),
               pltpu.VMEM((k, _B), jnp.float32),
               pltpu.VMEM_SHARED((_NP,), jnp.float32),
               pltpu.VMEM_SHARED((_NP,), jnp.float32),
               pltpu.SemaphoreType.DMA]
        ),
    )


def _agg2_body(k, e_ref, tab_ref, z_ref, out_ref, *scratch):
    sidx = scratch[:k]
    didx = scratch[k:2 * k]
    rows, acc, sem = scratch[2 * k:]
    steps = _EW // (_B * k)
    c = lax.axis_index("c")
    s = lax.axis_index("s")
    wid = c * _NS + s
    pltpu.sync_copy(z_ref.at[pl.ds(s * _RT, _RT)], acc.at[pl.ds(s * _RT, _RT)])
    plsc.subcore_barrier()

    def step(g, carry):
        _drain([pltpu.async_copy(e_ref.at[0, wid, g, j], sidx[j], sem)
                for j in range(k)]
               + [pltpu.async_copy(e_ref.at[1, wid, g, j], didx[j], sem)
                  for j in range(k)])
        _drain([pltpu.async_copy(tab_ref.at[sidx[j]], rows.at[j], sem)
                for j in range(k)])
        _drain([pltpu.async_copy(rows.at[j], acc.at[didx[j]], sem, add=True)
                for j in range(k)])
        return carry

    lax.fori_loop(0, steps, step, 0)
    plsc.subcore_barrier()
    pltpu.sync_copy(acc.at[pl.ds(s * _RT, _RT)],
                    out_ref.at[c, pl.ds(s * _RT, _RT)])


def _make_agg2_kernel(k=_K32):
    return pl.kernel(
        functools.partial(_agg2_body, k),
        out_type=jax.ShapeDtypeStruct((_NC, _NP, 32), jnp.float32),
        mesh=_mesh(),
        compiler_params=_SC_PARAMS,
        scratch_types=(
            [pltpu.VMEM((_B,), jnp.int32) for _ in range(2 * k)]
            + [pltpu.VMEM((k, _B, 32), jnp.float32),
               pltpu.VMEM_SHARED((_NP, 32), jnp.float32),
               pltpu.SemaphoreType.DMA]
        ),
    )


def _combine_body(pin_ref, pout_ref, hin_ref, hout_ref):
    hin_ref[...] = pin_ref[0] + pin_ref[1]
    hout_ref[...] = pout_ref[0] + pout_ref[1]


def _h1_body(pin_ref, pout_ref, w1t_ref, b1_ref, h1_ref):
    ain = (pin_ref[0] + pin_ref[1])[:, None]
    aout = (pout_ref[0] + pout_ref[1])[:, None]
    w1t = w1t_ref[...]
    h1 = ain * w1t[0:1, :] + aout * w1t[1:2, :] + b1_ref[...]
    h1_ref[...] = jnp.maximum(h1, 0.0)


_NB = 16


def _final_body(p_ref, w2t_ref, b2_ref, params_ref, pw1t_ref, pb1_ref,
                pw2t_ref, pb2_ref, fw1t_ref, fb1_ref, fw2t_ref, fb2_ref,
                ge_ref, pe_ref, m_ref, acc_ref):
    i = pl.program_id(0)
    agg = p_ref[0] + p_ref[1]
    h2 = jnp.dot(agg, w2t_ref[...],
                 preferred_element_type=jnp.float32) + b2_ref[...]
    r = jnp.maximum(h2, 0.0)
    row = lax.broadcasted_iota(jnp.int32, (_RT, 1), 0) + i * _RT
    r = jnp.where(row < _N, r, 0.0)
    part = jnp.sum(r, axis=0, keepdims=True)

    @pl.when(i == 0)
    def _():
        acc_ref[...] = part

    @pl.when(i > 0)
    def _():
        acc_ref[...] += part

    @pl.when(i == _NB - 1)
    def _():
        ge = acc_ref[...]
        ph = jnp.maximum(jnp.dot(params_ref[...], pw1t_ref[...],
                                 preferred_element_type=jnp.float32)
                         + pb1_ref[...], 0.0)
        pe = jnp.dot(ph, pw2t_ref[...],
                     preferred_element_type=jnp.float32) + pb2_ref[...]
        cat = jnp.concatenate([ge, pe], axis=1)
        m = jnp.maximum(jnp.dot(cat, fw1t_ref[...],
                                preferred_element_type=jnp.float32)
                        + fb1_ref[...], 0.0)
        m_ref[...] = jnp.dot(m, fw2t_ref[...],
                             preferred_element_type=jnp.float32) + fb2_ref[...]
        ge_ref[...] = ge
        pe_ref[...] = pe


def _full(shape):
    return pl.BlockSpec(shape, lambda i: tuple(0 for _ in shape))


def _make_final_call():
    f32 = jnp.float32
    return pl.pallas_call(
        _final_body,
        grid=(_NB,),
        in_specs=[
            pl.BlockSpec((_NC, _RT, 32), lambda i: (0, i, 0)),
            _full((32, 32)), _full((1, 32)), _full((1, 16)),
            _full((16, 64)), _full((1, 64)), _full((64, 32)), _full((1, 32)),
            _full((64, 64)), _full((1, 64)), _full((64, 4)), _full((1, 4)),
        ],
        out_specs=[_full((1, 32)), _full((1, 32)), _full((1, 4))],
        out_shape=[jax.ShapeDtypeStruct((1, 32), f32),
                   jax.ShapeDtypeStruct((1, 32), f32),
                   jax.ShapeDtypeStruct((1, 4), f32)],
        scratch_shapes=[pltpu.VMEM((1, 32), f32)],
    )


def kernel(edge_index, params, W1, b1, W2, b2, pW1, pb1, pW2, pb2,
           fW1, fb1, fW2, fb2):
    f32 = jnp.float32
    er2 = edge_index.reshape(2, _NW, _EW // (_B * _K2), _K2, _B)
    er32 = edge_index.reshape(2, _NW, _EW // (_B * _K32), _K32, _B)
    z1 = jnp.zeros((_NP,), f32)
    z32 = jnp.zeros((_NP, 32), f32)
    ones = jnp.ones((_B,), f32)

    din_p, dout_p = _make_deg_kernel()(er2, ones, z1)
    hin, hout = pl.pallas_call(
        _combine_body,
        out_shape=(jax.ShapeDtypeStruct((_NP,), f32),
                   jax.ShapeDtypeStruct((_NP,), f32)),
    )(din_p, dout_p)
    ain_p, aout_p = _make_agg1_kernel()(er2, hin, hout, z1)
    h1 = pl.pallas_call(
        _h1_body, out_shape=jax.ShapeDtypeStruct((_NP, 32), f32),
    )(ain_p, aout_p, W1.T, b1.reshape(1, 32))
    agg2_p = _make_agg2_kernel()(er32, h1, z32)
    ge, pe, metrics = _make_final_call()(
        agg2_p, W2.T, b2.reshape(1, 32), params, pW1.T, pb1.reshape(1, 64),
        pW2.T, pb2.reshape(1, 32), fW1.T, fb1.reshape(1, 64), fW2.T,
        fb2.reshape(1, 4))
    return (ge, pe, metrics)


# speedup vs baseline: 27.0079x; 1.0916x over previous
import functools

import jax
import jax.numpy as jnp
from jax import lax
from jax.experimental import pallas as pl
from jax.experimental.pallas import tpu as pltpu
from jax.experimental.pallas import tpu_sc as plsc

_N = 50000
_E = 1600000
_NC = 2
_NS = 16
_NW = _NC * _NS
_EW = _E // _NW
_B2, _KK2 = 80, 25
_B3, _KK3 = 40, 5
_NP = 50048
_RT = _NP // _NS


def _mesh():
    return plsc.VectorSubcoreMesh(core_axis_name="c", subcore_axis_name="s",
                                  num_cores=_NC, num_subcores=_NS)


_SC_PARAMS = pltpu.CompilerParams(use_tc_tiling_on_sc=False)


def _pipeline(steps, load_idx, fire_g, wait_g, fire_s, wait_s):
    load_idx(0, 0)
    fire_g(0)
    for c in (0, 1):
        a, y = c % 3, (c + 1) % 3
        load_idx(y, c + 1)
        fire_g(y)
        wait_g(a)
        fire_s(a)

    def triple(i, carry):
        base = 2 + 3 * i
        for o in range(3):
            a, y = (2 + o) % 3, o % 3
            wait_s(y)
            load_idx(y, base + o + 1)
            fire_g(y)
            wait_g(a)
            fire_s(a)
        return carry

    lax.fori_loop(0, (steps - 4) // 3, triple, 0)
    c = steps - 2
    wait_s(0)
    load_idx(0, c + 1)
    fire_g(0)
    wait_g(2)
    fire_s(2)
    wait_g(0)
    fire_s(0)
    for cc in (steps - 3, steps - 2, steps - 1):
        wait_s(cc % 3)


def _deg_body(k, e_ref, ones_ref, z_ref, oin_ref, oout_ref, *scratch):
    didx = scratch[0:3]
    sidx = scratch[3:6]
    ones, acc_i, acc_o = scratch[6:9]
    sems = scratch[9:12]
    steps = _EW // (_B2 * k)
    c = lax.axis_index("c")
    s = lax.axis_index("s")
    wid = c * _NS + s
    pltpu.sync_copy(z_ref.at[pl.ds(s * _RT, _RT)], acc_i.at[pl.ds(s * _RT, _RT)])
    pltpu.sync_copy(z_ref.at[pl.ds(s * _RT, _RT)], acc_o.at[pl.ds(s * _RT, _RT)])
    pltpu.sync_copy(ones_ref, ones)
    plsc.subcore_barrier()

    def load_idx(st, g):
        pltpu.sync_copy(e_ref.at[1, wid, g], didx[st])
        pltpu.sync_copy(e_ref.at[0, wid, g], sidx[st])

    def fire_s(st):
        for j in range(k):
            pltpu.async_copy(ones, acc_i.at[didx[st].at[j]], sems[st],
                             add=True)
            pltpu.async_copy(ones, acc_o.at[sidx[st].at[j]], sems[st],
                             add=True)

    def wait_s(st):
        for j in range(k):
            pltpu.make_async_copy(ones, acc_i.at[didx[st].at[j]],
                                  sems[st]).wait()
            pltpu.make_async_copy(ones, acc_o.at[sidx[st].at[j]],
                                  sems[st]).wait()

    _pipeline(steps, load_idx, lambda st: None, lambda st: None,
              fire_s, wait_s)
    plsc.subcore_barrier()
    pltpu.sync_copy(acc_i.at[pl.ds(s * _RT, _RT)],
                    oin_ref.at[c, pl.ds(s * _RT, _RT)])
    pltpu.sync_copy(acc_o.at[pl.ds(s * _RT, _RT)],
                    oout_ref.at[c, pl.ds(s * _RT, _RT)])


def _make_deg_kernel(k=_KK2):
    return pl.kernel(
        functools.partial(_deg_body, k),
        out_type=(jax.ShapeDtypeStruct((_NC, _NP), jnp.float32),
                  jax.ShapeDtypeStruct((_NC, _NP), jnp.float32)),
        mesh=_mesh(),
        compiler_params=_SC_PARAMS,
        scratch_types=(
            [pltpu.VMEM((k, _B2), jnp.int32) for _ in range(6)]
            + [pltpu.VMEM((_B2,), jnp.float32),
               pltpu.VMEM_SHARED((_NP,), jnp.float32),
               pltpu.VMEM_SHARED((_NP,), jnp.float32)]
            + [pltpu.SemaphoreType.DMA for _ in range(3)]
        ),
    )


def _agg1_body(k, e_ref, tin_ref, tout_ref, z_ref, oin_ref, oout_ref,
               *scratch):
    sidx = scratch[0:3]
    didx = scratch[3:6]
    rin = scratch[6:9]
    rout = scratch[9:12]
    acc_i, acc_o = scratch[12:14]
    semg = scratch[14:17]
    sems = scratch[17:20]
    steps = _EW // (_B2 * k)
    c = lax.axis_index("c")
    s = lax.axis_index("s")
    wid = c * _NS + s
    pltpu.sync_copy(z_ref.at[pl.ds(s * _RT, _RT)], acc_i.at[pl.ds(s * _RT, _RT)])
    pltpu.sync_copy(z_ref.at[pl.ds(s * _RT, _RT)], acc_o.at[pl.ds(s * _RT, _RT)])
    plsc.subcore_barrier()

    def load_idx(st, g):
        pltpu.sync_copy(e_ref.at[0, wid, g], sidx[st])
        pltpu.sync_copy(e_ref.at[1, wid, g], didx[st])

    def fire_g(st):
        for j in range(k):
            pltpu.async_copy(tin_ref.at[sidx[st].at[j]], rin[st].at[j],
                             semg[st])
            pltpu.async_copy(tout_ref.at[sidx[st].at[j]], rout[st].at[j],
                             semg[st])

    def wait_g(st):
        for j in range(k):
            pltpu.make_async_copy(tin_ref.at[sidx[st].at[j]], rin[st].at[j],
                                  semg[st]).wait()
            pltpu.make_async_copy(tout_ref.at[sidx[st].at[j]], rout[st].at[j],
                                  semg[st]).wait()

    def fire_s(st):
        for j in range(k):
            pltpu.async_copy(rin[st].at[j], acc_i.at[didx[st].at[j]],
                             sems[st], add=True)
            pltpu.async_copy(rout[st].at[j], acc_o.at[didx[st].at[j]],
                             sems[st], add=True)

    def wait_s(st):
        for j in range(k):
            pltpu.make_async_copy(rin[st].at[j], acc_i.at[didx[st].at[j]],
                                  sems[st]).wait()
            pltpu.make_async_copy(rout[st].at[j], acc_o.at[didx[st].at[j]],
                                  sems[st]).wait()

    _pipeline(steps, load_idx, fire_g, wait_g, fire_s, wait_s)
    plsc.subcore_barrier()
    pltpu.sync_copy(acc_i.at[pl.ds(s * _RT, _RT)],
                    oin_ref.at[c, pl.ds(s * _RT, _RT)])
    pltpu.sync_copy(acc_o.at[pl.ds(s * _RT, _RT)],
                    oout_ref.at[c, pl.ds(s * _RT, _RT)])


def _make_agg1_kernel(k=_KK2):
    return pl.kernel(
        functools.partial(_agg1_body, k),
        out_type=(jax.ShapeDtypeStruct((_NC, _NP), jnp.float32),
                  jax.ShapeDtypeStruct((_NC, _NP), jnp.float32)),
        mesh=_mesh(),
        compiler_params=_SC_PARAMS,
        scratch_types=(
            [pltpu.VMEM((k, _B2), jnp.int32) for _ in range(6)]
            + [pltpu.VMEM((k, _B2), jnp.float32) for _ in range(6)]
            + [pltpu.VMEM_SHARED((_NP,), jnp.float32),
               pltpu.VMEM_SHARED((_NP,), jnp.float32)]
            + [pltpu.SemaphoreType.DMA for _ in range(6)]
        ),
    )


def _agg2_body(k, e_ref, tab_ref, z_ref, out_ref, *scratch):
    sidx = scratch[0:3]
    didx = scratch[3:6]
    rows = scratch[6:9]
    acc = scratch[9]
    semg = scratch[10:13]
    sems = scratch[13:16]
    steps = _EW // (_B3 * k)
    c = lax.axis_index("c")
    s = lax.axis_index("s")
    wid = c * _NS + s
    pltpu.sync_copy(z_ref.at[pl.ds(s * _RT, _RT)], acc.at[pl.ds(s * _RT, _RT)])
    plsc.subcore_barrier()

    def load_idx(st, g):
        pltpu.sync_copy(e_ref.at[0, wid, g], sidx[st])
        pltpu.sync_copy(e_ref.at[1, wid, g], didx[st])

    def fire_g(st):
        for j in range(k):
            pltpu.async_copy(tab_ref.at[sidx[st].at[j]], rows[st].at[j],
                             semg[st])

    def wait_g(st):
        for j in range(k):
            pltpu.make_async_copy(tab_ref.at[sidx[st].at[j]], rows[st].at[j],
                                  semg[st]).wait()

    def fire_s(st):
        for j in range(k):
            pltpu.async_copy(rows[st].at[j], acc.at[didx[st].at[j]],
                             sems[st], add=True)

    def wait_s(st):
        for j in range(k):
            pltpu.make_async_copy(rows[st].at[j], acc.at[didx[st].at[j]],
                                  sems[st]).wait()

    _pipeline(steps, load_idx, fire_g, wait_g, fire_s, wait_s)
    plsc.subcore_barrier()
    pltpu.sync_copy(acc.at[pl.ds(s * _RT, _RT)],
                    out_ref.at[c, pl.ds(s * _RT, _RT)])


def _make_agg2_kernel(k=_KK3):
    return pl.kernel(
        functools.partial(_agg2_body, k),
        out_type=jax.ShapeDtypeStruct((_NC, _NP, 32), jnp.float32),
        mesh=_mesh(),
        compiler_params=_SC_PARAMS,
        scratch_types=(
            [pltpu.VMEM((k, _B3), jnp.int32) for _ in range(6)]
            + [pltpu.VMEM((k, _B3, 32), jnp.float32) for _ in range(3)]
            + [pltpu.VMEM_SHARED((_NP, 32), jnp.float32)]
            + [pltpu.SemaphoreType.DMA for _ in range(6)]
        ),
    )


def _combine_body(pin_ref, pout_ref, hin_ref, hout_ref):
    hin_ref[...] = pin_ref[0] + pin_ref[1]
    hout_ref[...] = pout_ref[0] + pout_ref[1]


def _h1_body(pin_ref, pout_ref, w1t_ref, b1_ref, h1_ref):
    ain = (pin_ref[0] + pin_ref[1])[:, None]
    aout = (pout_ref[0] + pout_ref[1])[:, None]
    w1t = w1t_ref[...]
    h1 = ain * w1t[0:1, :] + aout * w1t[1:2, :] + b1_ref[...]
    h1_ref[...] = jnp.maximum(h1, 0.0)


_NB = 16


def _final_body(p_ref, w2t_ref, b2_ref, params_ref, pw1t_ref, pb1_ref,
                pw2t_ref, pb2_ref, fw1t_ref, fb1_ref, fw2t_ref, fb2_ref,
                ge_ref, pe_ref, m_ref, acc_ref):
    i = pl.program_id(0)
    agg = p_ref[0] + p_ref[1]
    h2 = jnp.dot(agg, w2t_ref[...],
                 preferred_element_type=jnp.float32) + b2_ref[...]
    r = jnp.maximum(h2, 0.0)
    row = lax.broadcasted_iota(jnp.int32, (_RT, 1), 0) + i * _RT
    r = jnp.where(row < _N, r, 0.0)
    part = jnp.sum(r, axis=0, keepdims=True)

    @pl.when(i == 0)
    def _():
        acc_ref[...] = part

    @pl.when(i > 0)
    def _():
        acc_ref[...] += part

    @pl.when(i == _NB - 1)
    def _():
        ge = acc_ref[...]
        ph = jnp.maximum(jnp.dot(params_ref[...], pw1t_ref[...],
                                 preferred_element_type=jnp.float32)
                         + pb1_ref[...], 0.0)
        pe = jnp.dot(ph, pw2t_ref[...],
                     preferred_element_type=jnp.float32) + pb2_ref[...]
        cat = jnp.concatenate([ge, pe], axis=1)
        m = jnp.maximum(jnp.dot(cat, fw1t_ref[...],
                                preferred_element_type=jnp.float32)
                        + fb1_ref[...], 0.0)
        m_ref[...] = jnp.dot(m, fw2t_ref[...],
                             preferred_element_type=jnp.float32) + fb2_ref[...]
        ge_ref[...] = ge
        pe_ref[...] = pe


def _full(shape):
    return pl.BlockSpec(shape, lambda i: tuple(0 for _ in shape))


def _make_final_call():
    f32 = jnp.float32
    return pl.pallas_call(
        _final_body,
        grid=(_NB,),
        in_specs=[
            pl.BlockSpec((_NC, _RT, 32), lambda i: (0, i, 0)),
            _full((32, 32)), _full((1, 32)), _full((1, 16)),
            _full((16, 64)), _full((1, 64)), _full((64, 32)), _full((1, 32)),
            _full((64, 64)), _full((1, 64)), _full((64, 4)), _full((1, 4)),
        ],
        out_specs=[_full((1, 32)), _full((1, 32)), _full((1, 4))],
        out_shape=[jax.ShapeDtypeStruct((1, 32), f32),
                   jax.ShapeDtypeStruct((1, 32), f32),
                   jax.ShapeDtypeStruct((1, 4), f32)],
        scratch_shapes=[pltpu.VMEM((1, 32), f32)],
    )


def kernel(edge_index, params, W1, b1, W2, b2, pW1, pb1, pW2, pb2,
           fW1, fb1, fW2, fb2):
    f32 = jnp.float32
    er2 = edge_index.reshape(2, _NW, _EW // (_B2 * _KK2), _KK2, _B2)
    er32 = edge_index.reshape(2, _NW, _EW // (_B3 * _KK3), _KK3, _B3)
    z1 = jnp.zeros((_NP,), f32)
    z32 = jnp.zeros((_NP, 32), f32)
    ones = jnp.ones((_B2,), f32)

    din_p, dout_p = _make_deg_kernel()(er2, ones, z1)
    hin, hout = pl.pallas_call(
        _combine_body,
        out_shape=(jax.ShapeDtypeStruct((_NP,), f32),
                   jax.ShapeDtypeStruct((_NP,), f32)),
    )(din_p, dout_p)
    ain_p, aout_p = _make_agg1_kernel()(er2, hin, hout, z1)
    h1 = pl.pallas_call(
        _h1_body, out_shape=jax.ShapeDtypeStruct((_NP, 32), f32),
    )(ain_p, aout_p, W1.T, b1.reshape(1, 32))
    agg2_p = _make_agg2_kernel()(er32, h1, z32)
    ge, pe, metrics = _make_final_call()(
        agg2_p, W2.T, b2.reshape(1, 32), params, pW1.T, pb1.reshape(1, 64),
        pW2.T, pb2.reshape(1, 32), fW1.T, fb1.reshape(1, 64), fW2.T,
        fb2.reshape(1, 4))
    return (ge, pe, metrics)


# speedup vs baseline: 27.5088x; 1.0185x over previous
import functools

import jax
import jax.numpy as jnp
from jax import lax
from jax.experimental import pallas as pl
from jax.experimental.pallas import tpu as pltpu
from jax.experimental.pallas import tpu_sc as plsc

_N = 50000
_E = 1600000
_NC = 2
_NS = 16
_NW = _NC * _NS
_EW = _E // _NW
_B2, _KK2 = 80, 25
_B3, _KK3 = 200, 1
_NP = 50176
_RT = _NP // _NS


def _mesh():
    return plsc.VectorSubcoreMesh(core_axis_name="c", subcore_axis_name="s",
                                  num_cores=_NC, num_subcores=_NS)


_SC_PARAMS = pltpu.CompilerParams(use_tc_tiling_on_sc=False)
_SC_PARAMS_NL = pltpu.CompilerParams(use_tc_tiling_on_sc=False,
                                     needs_layout_passes=False)


def _pipeline(steps, load_idx, fire_g, wait_g, fire_s, wait_s):
    load_idx(0, 0)
    fire_g(0)
    for c in (0, 1):
        a, y = c % 3, (c + 1) % 3
        load_idx(y, c + 1)
        fire_g(y)
        wait_g(a)
        fire_s(a)

    def triple(i, carry):
        base = 2 + 3 * i
        for o in range(3):
            a, y = (2 + o) % 3, o % 3
            wait_s(y)
            load_idx(y, base + o + 1)
            fire_g(y)
            wait_g(a)
            fire_s(a)
        return carry

    lax.fori_loop(0, (steps - 4) // 3, triple, 0)
    c = steps - 2
    wait_s(0)
    load_idx(0, c + 1)
    fire_g(0)
    wait_g(2)
    fire_s(2)
    wait_g(0)
    fire_s(0)
    for cc in (steps - 3, steps - 2, steps - 1):
        wait_s(cc % 3)


def _deg_body(k, e_ref, ones_ref, z_ref, oin_ref, oout_ref, *scratch):
    didx = scratch[0:3]
    sidx = scratch[3:6]
    ones, acc_i, acc_o = scratch[6:9]
    sems = scratch[9:12]
    steps = _EW // (_B2 * k)
    c = lax.axis_index("c")
    s = lax.axis_index("s")
    wid = c * _NS + s
    pltpu.sync_copy(z_ref.at[pl.ds(s * _RT, _RT)], acc_i.at[pl.ds(s * _RT, _RT)])
    pltpu.sync_copy(z_ref.at[pl.ds(s * _RT, _RT)], acc_o.at[pl.ds(s * _RT, _RT)])
    pltpu.sync_copy(ones_ref, ones)
    plsc.subcore_barrier()

    def load_idx(st, g):
        pltpu.sync_copy(e_ref.at[1, wid, g], didx[st])
        pltpu.sync_copy(e_ref.at[0, wid, g], sidx[st])

    def fire_s(st):
        for j in range(k):
            pltpu.async_copy(ones, acc_i.at[didx[st].at[j]], sems[st], add=True)
            pltpu.async_copy(ones, acc_o.at[sidx[st].at[j]], sems[st], add=True)

    def wait_s(st):
        for j in range(k):
            pltpu.make_async_copy(ones, acc_i.at[didx[st].at[j]],
                                  sems[st]).wait()
            pltpu.make_async_copy(ones, acc_o.at[sidx[st].at[j]],
                                  sems[st]).wait()

    _pipeline(steps, load_idx, lambda st: None, lambda st: None,
              fire_s, wait_s)
    plsc.subcore_barrier()
    pltpu.sync_copy(acc_i.at[pl.ds(s * _RT, _RT)],
                    oin_ref.at[c, pl.ds(s * _RT, _RT)])
    pltpu.sync_copy(acc_o.at[pl.ds(s * _RT, _RT)],
                    oout_ref.at[c, pl.ds(s * _RT, _RT)])


def _make_deg_kernel(k=_KK2):
    return pl.kernel(
        functools.partial(_deg_body, k),
        out_type=(jax.ShapeDtypeStruct((_NC, _NP), jnp.float32),
                  jax.ShapeDtypeStruct((_NC, _NP), jnp.float32)),
        mesh=_mesh(),
        compiler_params=_SC_PARAMS,
        scratch_types=(
            [pltpu.VMEM((k, _B2), jnp.int32) for _ in range(6)]
            + [pltpu.VMEM((_B2,), jnp.float32),
               pltpu.VMEM_SHARED((_NP,), jnp.float32),
               pltpu.VMEM_SHARED((_NP,), jnp.float32)]
            + [pltpu.SemaphoreType.DMA for _ in range(3)]
        ),
    )


def _agg1_body(k, e_ref, tin_ref, tout_ref, z_ref, oin_ref, oout_ref,
               *scratch):
    sidx = scratch[0:3]
    didx = scratch[3:6]
    rin = scratch[6:9]
    rout = scratch[9:12]
    acc_i, acc_o = scratch[12:14]
    semg = scratch[14:17]
    sems = scratch[17:20]
    steps = _EW // (_B2 * k)
    c = lax.axis_index("c")
    s = lax.axis_index("s")
    wid = c * _NS + s
    pltpu.sync_copy(z_ref.at[pl.ds(s * _RT, _RT)], acc_i.at[pl.ds(s * _RT, _RT)])
    pltpu.sync_copy(z_ref.at[pl.ds(s * _RT, _RT)], acc_o.at[pl.ds(s * _RT, _RT)])
    plsc.subcore_barrier()

    def load_idx(st, g):
        pltpu.sync_copy(e_ref.at[0, wid, g], sidx[st])
        pltpu.sync_copy(e_ref.at[1, wid, g], didx[st])

    def fire_g(st):
        for j in range(k):
            pltpu.async_copy(tin_ref.at[sidx[st].at[j]], rin[st].at[j],
                             semg[st])
            pltpu.async_copy(tout_ref.at[sidx[st].at[j]], rout[st].at[j],
                             semg[st])

    def wait_g(st):
        for j in range(k):
            pltpu.make_async_copy(tin_ref.at[sidx[st].at[j]], rin[st].at[j],
                                  semg[st]).wait()
            pltpu.make_async_copy(tout_ref.at[sidx[st].at[j]], rout[st].at[j],
                                  semg[st]).wait()

    def fire_s(st):
        for j in range(k):
            pltpu.async_copy(rin[st].at[j], acc_i.at[didx[st].at[j]],
                             sems[st], add=True)
            pltpu.async_copy(rout[st].at[j], acc_o.at[didx[st].at[j]],
                             sems[st], add=True)

    def wait_s(st):
        for j in range(k):
            pltpu.make_async_copy(rin[st].at[j], acc_i.at[didx[st].at[j]],
                                  sems[st]).wait()
            pltpu.make_async_copy(rout[st].at[j], acc_o.at[didx[st].at[j]],
                                  sems[st]).wait()

    _pipeline(steps, load_idx, fire_g, wait_g, fire_s, wait_s)
    plsc.subcore_barrier()
    pltpu.sync_copy(acc_i.at[pl.ds(s * _RT, _RT)],
                    oin_ref.at[c, pl.ds(s * _RT, _RT)])
    pltpu.sync_copy(acc_o.at[pl.ds(s * _RT, _RT)],
                    oout_ref.at[c, pl.ds(s * _RT, _RT)])


def _make_agg1_kernel(k=_KK2):
    return pl.kernel(
        functools.partial(_agg1_body, k),
        out_type=(jax.ShapeDtypeStruct((_NC, _NP), jnp.float32),
                  jax.ShapeDtypeStruct((_NC, _NP), jnp.float32)),
        mesh=_mesh(),
        compiler_params=_SC_PARAMS,
        scratch_types=(
            [pltpu.VMEM((k, _B2), jnp.int32) for _ in range(6)]
            + [pltpu.VMEM((k, _B2), jnp.float32) for _ in range(6)]
            + [pltpu.VMEM_SHARED((_NP,), jnp.float32),
               pltpu.VMEM_SHARED((_NP,), jnp.float32)]
            + [pltpu.SemaphoreType.DMA for _ in range(6)]
        ),
    )


def _agg2_body(k, e_ref, tab_ref, z_ref, out_ref, *scratch):
    sidx = scratch[0:3]
    didx = scratch[3:6]
    rows = scratch[6:9]
    acc = scratch[9]
    semg = scratch[10:13]
    sems = scratch[13:16]
    steps = _EW // (_B3 * k)
    c = lax.axis_index("c")
    s = lax.axis_index("s")
    wid = c * _NS + s
    pltpu.sync_copy(z_ref.at[pl.ds(s * _RT, _RT)], acc.at[pl.ds(s * _RT, _RT)])
    plsc.subcore_barrier()

    def load_idx(st, g):
        pltpu.sync_copy(e_ref.at[0, wid, g], sidx[st])
        pltpu.sync_copy(e_ref.at[1, wid, g], didx[st])

    def fire_g(st):
        for j in range(k):
            pltpu.async_copy(tab_ref.at[sidx[st].at[j]], rows[st].at[j],
                             semg[st])

    def wait_g(st):
        for j in range(k):
            pltpu.make_async_copy(tab_ref.at[sidx[st].at[j]], rows[st].at[j],
                                  semg[st]).wait()

    def fire_s(st):
        for j in range(k):
            pltpu.async_copy(rows[st].at[j], acc.at[didx[st].at[j]],
                             sems[st], add=True)

    def wait_s(st):
        for j in range(k):
            pltpu.make_async_copy(rows[st].at[j], acc.at[didx[st].at[j]],
                                  sems[st]).wait()

    _pipeline(steps, load_idx, fire_g, wait_g, fire_s, wait_s)
    plsc.subcore_barrier()
    pltpu.sync_copy(acc.at[pl.ds(s * _RT, _RT)],
                    out_ref.at[c, pl.ds(s * _RT, _RT)])


def _make_agg2_kernel(k=_KK3):
    return pl.kernel(
        functools.partial(_agg2_body, k),
        out_type=jax.ShapeDtypeStruct((_NC, _NP, 32), jnp.float32),
        mesh=_mesh(),
        compiler_params=_SC_PARAMS,
        scratch_types=(
            [pltpu.VMEM((k, _B3), jnp.int32) for _ in range(6)]
            + [pltpu.VMEM((k, _B3, 32), jnp.float32) for _ in range(3)]
            + [pltpu.VMEM_SHARED((_NP, 32), jnp.float32)]
            + [pltpu.SemaphoreType.DMA for _ in range(6)]
        ),
    )


def _combine_body(pin_ref, pout_ref, hin_ref, hout_ref):
    hin_ref[...] = pin_ref[0] + pin_ref[1]
    hout_ref[...] = pout_ref[0] + pout_ref[1]


def _h1_body(pin_ref, pout_ref, w1t_ref, b1_ref, h1_ref):
    ain = (pin_ref[0] + pin_ref[1])[:, None]
    aout = (pout_ref[0] + pout_ref[1])[:, None]
    w1t = w1t_ref[...]
    h1 = ain * w1t[0:1, :] + aout * w1t[1:2, :] + b1_ref[...]
    h1_ref[...] = jnp.maximum(h1, 0.0)


_NB = 16


def _final_body(p_ref, w2t_ref, b2_ref, params_ref, pw1t_ref, pb1_ref,
                pw2t_ref, pb2_ref, fw1t_ref, fb1_ref, fw2t_ref, fb2_ref,
                ge_ref, pe_ref, m_ref, acc_ref):
    i = pl.program_id(0)
    agg = p_ref[0] + p_ref[1]
    h2 = jnp.dot(agg, w2t_ref[...],
                 preferred_element_type=jnp.float32) + b2_ref[...]
    r = jnp.maximum(h2, 0.0)
    row = lax.broadcasted_iota(jnp.int32, (_RT, 1), 0) + i * _RT
    r = jnp.where(row < _N, r, 0.0)
    part = jnp.sum(r, axis=0, keepdims=True)

    @pl.when(i == 0)
    def _():
        acc_ref[...] = part

    @pl.when(i > 0)
    def _():
        acc_ref[...] += part

    @pl.when(i == _NB - 1)
    def _():
        ge = acc_ref[...]
        ph = jnp.maximum(jnp.dot(params_ref[...], pw1t_ref[...],
                                 preferred_element_type=jnp.float32)
                         + pb1_ref[...], 0.0)
        pe = jnp.dot(ph, pw2t_ref[...],
                     preferred_element_type=jnp.float32) + pb2_ref[...]
        cat = jnp.concatenate([ge, pe], axis=1)
        m = jnp.maximum(jnp.dot(cat, fw1t_ref[...],
                                preferred_element_type=jnp.float32)
                        + fb1_ref[...], 0.0)
        m_ref[...] = jnp.dot(m, fw2t_ref[...],
                             preferred_element_type=jnp.float32) + fb2_ref[...]
        ge_ref[...] = ge
        pe_ref[...] = pe


def _full(shape):
    return pl.BlockSpec(shape, lambda i: tuple(0 for _ in shape))


def _make_final_call():
    f32 = jnp.float32
    return pl.pallas_call(
        _final_body,
        grid=(_NB,),
        in_specs=[
            pl.BlockSpec((_NC, _RT, 32), lambda i: (0, i, 0)),
            _full((32, 32)), _full((1, 32)), _full((1, 16)),
            _full((16, 64)), _full((1, 64)), _full((64, 32)), _full((1, 32)),
            _full((64, 64)), _full((1, 64)), _full((64, 4)), _full((1, 4)),
        ],
        out_specs=[_full((1, 32)), _full((1, 32)), _full((1, 4))],
        out_shape=[jax.ShapeDtypeStruct((1, 32), f32),
                   jax.ShapeDtypeStruct((1, 32), f32),
                   jax.ShapeDtypeStruct((1, 4), f32)],
        scratch_shapes=[pltpu.VMEM((1, 32), f32)],
    )


def kernel(edge_index, params, W1, b1, W2, b2, pW1, pb1, pW2, pb2,
           fW1, fb1, fW2, fb2):
    f32 = jnp.float32
    er2 = edge_index.reshape(2, _NW, _EW // (_B2 * _KK2), _KK2, _B2)
    er32 = edge_index.reshape(2, _NW, _EW // (_B3 * _KK3), _KK3, _B3)
    z1 = jnp.zeros((_NP,), f32)
    z32 = jnp.zeros((_NP, 32), f32)
    ones = jnp.ones((_B2,), f32)

    din_p, dout_p = _make_deg_kernel()(er2, ones, z1)
    hin, hout = pl.pallas_call(
        _combine_body,
        out_shape=(jax.ShapeDtypeStruct((_NP,), f32),
                   jax.ShapeDtypeStruct((_NP,), f32)),
    )(din_p, dout_p)
    ain_p, aout_p = _make_agg1_kernel()(er2, hin, hout, z1)
    h1 = pl.pallas_call(
        _h1_body, out_shape=jax.ShapeDtypeStruct((_NP, 32), f32),
    )(ain_p, aout_p, W1.T, b1.reshape(1, 32))
    agg2_p = _make_agg2_kernel()(er32, h1, z32)
    ge, pe, metrics = _make_final_call()(
        agg2_p, W2.T, b2.reshape(1, 32), params, pW1.T, pb1.reshape(1, 64),
        pW2.T, pb2.reshape(1, 32), fW1.T, fb1.reshape(1, 64), fW2.T,
        fb2.reshape(1, 4))
    return (ge, pe, metrics)


# speedup vs baseline: 33.4919x; 1.2175x over previous
import functools

import jax
import jax.numpy as jnp
from jax import lax
from jax.experimental import pallas as pl
from jax.experimental.pallas import tpu as pltpu
from jax.experimental.pallas import tpu_sc as plsc

_N = 50000
_E = 1600000
_NC = 2
_NS = 16
_NW = _NC * _NS
_EW = _E // _NW
_B2, _KK2 = 80, 25
_B3, _KK3 = 200, 1
_NP = 50176
_RT = _NP // _NS


def _mesh():
    return plsc.VectorSubcoreMesh(core_axis_name="c", subcore_axis_name="s",
                                  num_cores=_NC, num_subcores=_NS)


_SC_PARAMS = pltpu.CompilerParams(use_tc_tiling_on_sc=False)
_SC_PARAMS_NL = pltpu.CompilerParams(use_tc_tiling_on_sc=False,
                                     needs_layout_passes=False)


def _pipeline(steps, load_idx, fire_g, wait_g, fire_s, wait_s):
    load_idx(0, 0)
    fire_g(0)
    for c in (0, 1):
        a, y = c % 3, (c + 1) % 3
        load_idx(y, c + 1)
        fire_g(y)
        wait_g(a)
        fire_s(a)

    def triple(i, carry):
        base = 2 + 3 * i
        for o in range(3):
            a, y = (2 + o) % 3, o % 3
            wait_s(y)
            load_idx(y, base + o + 1)
            fire_g(y)
            wait_g(a)
            fire_s(a)
        return carry

    lax.fori_loop(0, (steps - 4) // 3, triple, 0)
    c = steps - 2
    wait_s(0)
    load_idx(0, c + 1)
    fire_g(0)
    wait_g(2)
    fire_s(2)
    wait_g(0)
    fire_s(0)
    for cc in (steps - 3, steps - 2, steps - 1):
        wait_s(cc % 3)


def _drain(descs):
    for d in descs:
        d.wait()


def _deg_body(k, e_ref, ones_ref, z_ref, oin_ref, oout_ref, *scratch):
    didx = scratch[0:3]
    sidx = scratch[3:6]
    ones, acc_i, acc_o = scratch[6:9]
    sems = scratch[9:12]
    steps = _EW // (_B2 * k)
    c = lax.axis_index("c")
    s = lax.axis_index("s")
    wid = c * _NS + s
    pltpu.sync_copy(z_ref.at[pl.ds(s * _RT, _RT)], acc_i.at[pl.ds(s * _RT, _RT)])
    pltpu.sync_copy(z_ref.at[pl.ds(s * _RT, _RT)], acc_o.at[pl.ds(s * _RT, _RT)])
    pltpu.sync_copy(ones_ref, ones)
    plsc.subcore_barrier()

    def load_idx(st, g):
        pltpu.sync_copy(e_ref.at[1, wid, g], didx[st])
        pltpu.sync_copy(e_ref.at[0, wid, g], sidx[st])

    def fire_s(st):
        for j in range(k):
            pltpu.async_copy(ones, acc_i.at[didx[st].at[j]], sems[st], add=True)
            pltpu.async_copy(ones, acc_o.at[sidx[st].at[j]], sems[st], add=True)

    def wait_s(st):
        for j in range(k):
            pltpu.make_async_copy(ones, acc_i.at[didx[st].at[j]],
                                  sems[st]).wait()
            pltpu.make_async_copy(ones, acc_o.at[sidx[st].at[j]],
                                  sems[st]).wait()

    _pipeline(steps, load_idx, lambda st: None, lambda st: None,
              fire_s, wait_s)
    plsc.subcore_barrier()
    pltpu.sync_copy(acc_i.at[pl.ds(s * _RT, _RT)],
                    oin_ref.at[c, pl.ds(s * _RT, _RT)])
    pltpu.sync_copy(acc_o.at[pl.ds(s * _RT, _RT)],
                    oout_ref.at[c, pl.ds(s * _RT, _RT)])


def _make_deg_kernel(k=_KK2):
    return pl.kernel(
        functools.partial(_deg_body, k),
        out_type=(jax.ShapeDtypeStruct((_NC, _NP), jnp.float32),
                  jax.ShapeDtypeStruct((_NC, _NP), jnp.float32)),
        mesh=_mesh(),
        compiler_params=_SC_PARAMS,
        scratch_types=(
            [pltpu.VMEM((k, _B2), jnp.int32) for _ in range(6)]
            + [pltpu.VMEM((_B2,), jnp.float32),
               pltpu.VMEM_SHARED((_NP,), jnp.float32),
               pltpu.VMEM_SHARED((_NP,), jnp.float32)]
            + [pltpu.SemaphoreType.DMA for _ in range(3)]
        ),
    )


_DI = 28
_DJ = _RT // _DI


def _agg1_body(k, e_ref, dpi_ref, dpo_ref, iidx_ref,
               z_ref, oin_ref, oout_ref, *scratch):
    sidx = scratch[0:3]
    didx = scratch[3:6]
    rin = scratch[6:9]
    rout = scratch[9:12]
    tbl_i, tbl_o, acc_i, acc_o, pbuf, iv = scratch[12:18]
    semg = scratch[18:21]
    sems = scratch[21:24]
    steps = _EW // (_B2 * k)
    c = lax.axis_index("c")
    s = lax.axis_index("s")
    wid = c * _NS + s
    sl = pl.ds(s * _RT, _RT)
    pltpu.sync_copy(dpi_ref.at[0, sl], tbl_i.at[sl])
    pltpu.sync_copy(dpo_ref.at[0, sl], tbl_o.at[sl])
    pltpu.sync_copy(iidx_ref.at[s], iv)
    _drain([pltpu.async_copy(
        dpi_ref.at[1, pl.ds(s * _RT + j * _DJ, _DJ)], pbuf.at[j], semg[0])
        for j in range(_DI)])
    for j in range(_DI):
        pltpu.sync_copy(pbuf.at[j], tbl_i.at[iv.at[j]], add=True)
    _drain([pltpu.async_copy(
        dpo_ref.at[1, pl.ds(s * _RT + j * _DJ, _DJ)], pbuf.at[j], semg[0])
        for j in range(_DI)])
    for j in range(_DI):
        pltpu.sync_copy(pbuf.at[j], tbl_o.at[iv.at[j]], add=True)
    pltpu.sync_copy(z_ref.at[sl], acc_i.at[sl])
    pltpu.sync_copy(z_ref.at[sl], acc_o.at[sl])
    plsc.subcore_barrier()

    def load_idx(st, g):
        pltpu.sync_copy(e_ref.at[0, wid, g], sidx[st])
        pltpu.sync_copy(e_ref.at[1, wid, g], didx[st])

    def fire_g(st):
        for j in range(k):
            pltpu.async_copy(tbl_i.at[sidx[st].at[j]], rin[st].at[j],
                             semg[st])
            pltpu.async_copy(tbl_o.at[sidx[st].at[j]], rout[st].at[j],
                             semg[st])

    def wait_g(st):
        for j in range(k):
            pltpu.make_async_copy(tbl_i.at[sidx[st].at[j]], rin[st].at[j],
                                  semg[st]).wait()
            pltpu.make_async_copy(tbl_o.at[sidx[st].at[j]], rout[st].at[j],
                                  semg[st]).wait()

    def fire_s(st):
        for j in range(k):
            pltpu.async_copy(rin[st].at[j], acc_i.at[didx[st].at[j]],
                             sems[st], add=True)
            pltpu.async_copy(rout[st].at[j], acc_o.at[didx[st].at[j]],
                             sems[st], add=True)

    def wait_s(st):
        for j in range(k):
            pltpu.make_async_copy(rin[st].at[j], acc_i.at[didx[st].at[j]],
                                  sems[st]).wait()
            pltpu.make_async_copy(rout[st].at[j], acc_o.at[didx[st].at[j]],
                                  sems[st]).wait()

    _pipeline(steps, load_idx, fire_g, wait_g, fire_s, wait_s)
    plsc.subcore_barrier()
    pltpu.sync_copy(acc_i.at[sl], oin_ref.at[c, sl])
    pltpu.sync_copy(acc_o.at[sl], oout_ref.at[c, sl])


def _make_agg1_kernel(k=_KK2):
    return pl.kernel(
        functools.partial(_agg1_body, k),
        out_type=(jax.ShapeDtypeStruct((_NC, _NP), jnp.float32),
                  jax.ShapeDtypeStruct((_NC, _NP), jnp.float32)),
        mesh=_mesh(),
        compiler_params=_SC_PARAMS,
        scratch_types=(
            [pltpu.VMEM((k, _B2), jnp.int32) for _ in range(6)]
            + [pltpu.VMEM((k, _B2), jnp.float32) for _ in range(6)]
            + [pltpu.VMEM_SHARED((_NP,), jnp.float32),
               pltpu.VMEM_SHARED((_NP,), jnp.float32),
               pltpu.VMEM_SHARED((_NP,), jnp.float32),
               pltpu.VMEM_SHARED((_NP,), jnp.float32),
               pltpu.VMEM((_DI, _DJ), jnp.float32),
               pltpu.VMEM((_DI, _DJ), jnp.int32)]
            + [pltpu.SemaphoreType.DMA for _ in range(6)]
        ),
    )


def _agg2_body(k, e_ref, ai_ref, ao_ref, w_ref, z_ref, out_ref, h1o_ref, *scratch):
    sidx = scratch[0:3]
    didx = scratch[3:6]
    rows = scratch[6:9]
    acc, ab, ob, wb = scratch[9:13]
    semg = scratch[13:16]
    sems = scratch[16:19]
    steps = _EW // (_B3 * k)
    c = lax.axis_index("c")
    s = lax.axis_index("s")
    wid = c * _NS + s
    sl = pl.ds(s * _RT, _RT)
    pltpu.sync_copy(w_ref, wb)
    w0a = wb[0, pl.ds(0, 16)]
    w0b = wb[0, pl.ds(16, 16)]
    w1a = wb[1, pl.ds(0, 16)]
    w1b = wb[1, pl.ds(16, 16)]
    b1a = wb[2, pl.ds(0, 16)]
    b1b = wb[2, pl.ds(16, 16)]
    rq = _RT // 4
    for q in range(4):
        qoff = s * _RT + q * rq
        pltpu.sync_copy(ai_ref.at[0, pl.ds(qoff, rq)], ab.at[0])
        pltpu.sync_copy(ai_ref.at[1, pl.ds(qoff, rq)], ab.at[1])
        pltpu.sync_copy(ao_ref.at[0, pl.ds(qoff, rq)], ab.at[2])
        pltpu.sync_copy(ao_ref.at[1, pl.ds(qoff, rq)], ab.at[3])

        def grp(g, carry, qoff=qoff):
            base = g * 16
            ainv = ab[0, pl.ds(base, 16)] + ab[1, pl.ds(base, 16)]
            aoutv = ab[2, pl.ds(base, 16)] + ab[3, pl.ds(base, 16)]
            for n in range(16):
                ain = ainv[n]
                aout = aoutv[n]
                ob[n, pl.ds(0, 16)] = jnp.maximum(
                    ain * w0a + aout * w1a + b1a, 0.0)
                ob[n, pl.ds(16, 16)] = jnp.maximum(
                    ain * w0b + aout * w1b + b1b, 0.0)
            pltpu.sync_copy(ob, h1o_ref.at[c, pl.ds(qoff + base, 16)])
            return carry

        lax.fori_loop(0, rq // 16, grp, 0)
    pltpu.sync_copy(z_ref.at[sl], acc.at[sl])
    plsc.subcore_barrier()

    tab = h1o_ref.at[c]

    def load_idx(st, g):
        pltpu.sync_copy(e_ref.at[0, wid, g], sidx[st])
        pltpu.sync_copy(e_ref.at[1, wid, g], didx[st])

    def fire_g(st):
        for j in range(k):
            pltpu.async_copy(tab.at[sidx[st].at[j]], rows[st].at[j],
                             semg[st])

    def wait_g(st):
        for j in range(k):
            pltpu.make_async_copy(tab.at[sidx[st].at[j]], rows[st].at[j],
                                  semg[st]).wait()

    def fire_s(st):
        for j in range(k):
            pltpu.async_copy(rows[st].at[j], acc.at[didx[st].at[j]],
                             sems[st], add=True)

    def wait_s(st):
        for j in range(k):
            pltpu.make_async_copy(rows[st].at[j], acc.at[didx[st].at[j]],
                                  sems[st]).wait()

    _pipeline(steps, load_idx, fire_g, wait_g, fire_s, wait_s)
    plsc.subcore_barrier()
    pltpu.sync_copy(acc.at[sl], out_ref.at[c, sl])


def _make_agg2_kernel(k=_KK3):
    return pl.kernel(
        functools.partial(_agg2_body, k),
        out_type=(jax.ShapeDtypeStruct((_NC, _NP, 32), jnp.float32),
                  jax.ShapeDtypeStruct((_NC, _NP, 32), jnp.float32)),
        mesh=_mesh(),
        compiler_params=_SC_PARAMS,
        scratch_types=(
            [pltpu.VMEM((k, _B3), jnp.int32) for _ in range(6)]
            + [pltpu.VMEM((k, _B3, 32), jnp.float32) for _ in range(3)]
            + [pltpu.VMEM_SHARED((_NP, 32), jnp.float32),
               pltpu.VMEM((4, _RT // 4), jnp.float32),
               pltpu.VMEM((16, 32), jnp.float32),
               pltpu.VMEM((3, 32), jnp.float32)]
            + [pltpu.SemaphoreType.DMA for _ in range(6)]
        ),
    )


_NB = 16


def _final_body(p_ref, w2t_ref, b2_ref, params_ref, pw1t_ref, pb1_ref,
                pw2t_ref, pb2_ref, fw1t_ref, fb1_ref, fw2t_ref, fb2_ref,
                ge_ref, pe_ref, m_ref, acc_ref):
    i = pl.program_id(0)
    agg = p_ref[0] + p_ref[1]
    h2 = jnp.dot(agg, w2t_ref[...],
                 preferred_element_type=jnp.float32) + b2_ref[...]
    r = jnp.maximum(h2, 0.0)
    row = lax.broadcasted_iota(jnp.int32, (_RT, 1), 0) + i * _RT
    r = jnp.where(row < _N, r, 0.0)
    part = jnp.sum(r, axis=0, keepdims=True)

    @pl.when(i == 0)
    def _():
        acc_ref[...] = part

    @pl.when(i > 0)
    def _():
        acc_ref[...] += part

    @pl.when(i == _NB - 1)
    def _():
        ge = acc_ref[...]
        ph = jnp.maximum(jnp.dot(params_ref[...], pw1t_ref[...],
                                 preferred_element_type=jnp.float32)
                         + pb1_ref[...], 0.0)
        pe = jnp.dot(ph, pw2t_ref[...],
                     preferred_element_type=jnp.float32) + pb2_ref[...]
        cat = jnp.concatenate([ge, pe], axis=1)
        m = jnp.maximum(jnp.dot(cat, fw1t_ref[...],
                                preferred_element_type=jnp.float32)
                        + fb1_ref[...], 0.0)
        m_ref[...] = jnp.dot(m, fw2t_ref[...],
                             preferred_element_type=jnp.float32) + fb2_ref[...]
        ge_ref[...] = ge
        pe_ref[...] = pe


def _full(shape):
    return pl.BlockSpec(shape, lambda i: tuple(0 for _ in shape))


def _make_final_call():
    f32 = jnp.float32
    return pl.pallas_call(
        _final_body,
        grid=(_NB,),
        in_specs=[
            pl.BlockSpec((_NC, _RT, 32), lambda i: (0, i, 0)),
            _full((32, 32)), _full((1, 32)), _full((1, 16)),
            _full((16, 64)), _full((1, 64)), _full((64, 32)), _full((1, 32)),
            _full((64, 64)), _full((1, 64)), _full((64, 4)), _full((1, 4)),
        ],
        out_specs=[_full((1, 32)), _full((1, 32)), _full((1, 4))],
        out_shape=[jax.ShapeDtypeStruct((1, 32), f32),
                   jax.ShapeDtypeStruct((1, 32), f32),
                   jax.ShapeDtypeStruct((1, 4), f32)],
        scratch_shapes=[pltpu.VMEM((1, 32), f32)],
    )


def kernel(edge_index, params, W1, b1, W2, b2, pW1, pb1, pW2, pb2,
           fW1, fb1, fW2, fb2):
    f32 = jnp.float32
    er2 = edge_index.reshape(2, _NW, _EW // (_B2 * _KK2), _KK2, _B2)
    er32 = edge_index.reshape(2, _NW, _EW // (_B3 * _KK3), _KK3, _B3)
    z1 = jnp.zeros((_NP,), f32)
    z32 = jnp.zeros((_NP, 32), f32)
    ones = jnp.ones((_B2,), f32)
    iidx = jnp.arange(_NP, dtype=jnp.int32).reshape(_NS, _DI, _DJ)
    wcat = jnp.stack([W1[:, 0], W1[:, 1], b1], 0)

    din_p, dout_p = _make_deg_kernel()(er2, ones, z1)
    ain_p, aout_p = _make_agg1_kernel()(er2, din_p, dout_p, iidx, z1)
    agg2_p, _h1o = _make_agg2_kernel()(er32, ain_p, aout_p, wcat, z32)
    ge, pe, metrics = _make_final_call()(
        agg2_p, W2.T, b2.reshape(1, 32), params, pW1.T, pb1.reshape(1, 64),
        pW2.T, pb2.reshape(1, 32), fW1.T, fb1.reshape(1, 64), fW2.T,
        fb2.reshape(1, 4))
    return (ge, pe, metrics)


# speedup vs baseline: 35.1774x; 1.0503x over previous
import functools

import jax
import jax.numpy as jnp
from jax import lax
from jax.experimental import pallas as pl
from jax.experimental.pallas import tpu as pltpu
from jax.experimental.pallas import tpu_sc as plsc

_N = 50000
_E = 1600000
_NC = 2
_NS = 16
_NW = _NC * _NS
_EW = _E // _NW
_B2, _KK2 = 80, 25
_B3, _KK3 = 200, 1
_NP = 50176
_RT = _NP // _NS


def _mesh():
    return plsc.VectorSubcoreMesh(core_axis_name="c", subcore_axis_name="s",
                                  num_cores=_NC, num_subcores=_NS)


_SC_PARAMS = pltpu.CompilerParams(use_tc_tiling_on_sc=False)
_SC_PARAMS_NL = pltpu.CompilerParams(use_tc_tiling_on_sc=False,
                                     needs_layout_passes=False)


def _pipeline(steps, load_idx, fire_g, wait_g, fire_s, wait_s):
    load_idx(0, 0)
    fire_g(0)
    for c in (0, 1):
        a, y = c % 3, (c + 1) % 3
        load_idx(y, c + 1)
        fire_g(y)
        wait_g(a)
        fire_s(a)

    def triple(i, carry):
        base = 2 + 3 * i
        for o in range(3):
            a, y = (2 + o) % 3, o % 3
            wait_s(y)
            load_idx(y, base + o + 1)
            fire_g(y)
            wait_g(a)
            fire_s(a)
        return carry

    lax.fori_loop(0, (steps - 4) // 3, triple, 0)
    c = steps - 2
    wait_s(0)
    load_idx(0, c + 1)
    fire_g(0)
    wait_g(2)
    fire_s(2)
    wait_g(0)
    fire_s(0)
    for cc in (steps - 3, steps - 2, steps - 1):
        wait_s(cc % 3)


def _drain(descs):
    for d in descs:
        d.wait()


def _pipeline4(steps, fire_idx, wait_idx, fire_g, wait_g, fire_s, wait_s):
    fire_idx(0, 0)
    wait_idx(0)
    fire_g(0, 0)
    fire_idx(1, 1)
    for c in (0, 1):
        a3, y3 = c % 3, (c + 1) % 3
        wait_idx((c + 1) % 4)
        fire_g(y3, (c + 1) % 4)
        fire_idx((c + 2) % 4, c + 2)
        wait_g(a3)
        fire_s(a3, c % 4)
    r = (steps - 2) % 12
    if r < 2:
        r += 12
    t = (steps - 2 - r) // 12

    def body(i, carry):
        base = 2 + 12 * i
        for o in range(12):
            a3, y3 = (2 + o) % 3, (3 + o) % 3
            wait_s(y3)
            wait_idx((3 + o) % 4)
            fire_g(y3, (3 + o) % 4)
            fire_idx((4 + o) % 4, base + o + 2)
            wait_g(a3)
            fire_s(a3, (2 + o) % 4)
        return carry

    lax.fori_loop(0, t, body, 0)
    for c in range(2 + 12 * t, steps):
        a3, y3 = c % 3, (c + 1) % 3
        if c + 1 <= steps - 1:
            wait_s(y3)
            wait_idx((c + 1) % 4)
            fire_g(y3, (c + 1) % 4)
        if c + 2 <= steps - 1:
            fire_idx((c + 2) % 4, c + 2)
        wait_g(a3)
        fire_s(a3, c % 4)
    for cc in (steps - 3, steps - 2, steps - 1):
        wait_s(cc % 3)


def _deg_body(k, e_ref, ones_ref, z_ref, oin_ref, oout_ref, *scratch):
    didx = scratch[0:3]
    sidx = scratch[3:6]
    ones, acc_i, acc_o = scratch[6:9]
    sems = scratch[9:12]
    steps = _EW // (_B2 * k)
    c = lax.axis_index("c")
    s = lax.axis_index("s")
    wid = c * _NS + s
    pltpu.sync_copy(z_ref.at[pl.ds(s * _RT, _RT)], acc_i.at[pl.ds(s * _RT, _RT)])
    pltpu.sync_copy(z_ref.at[pl.ds(s * _RT, _RT)], acc_o.at[pl.ds(s * _RT, _RT)])
    pltpu.sync_copy(ones_ref, ones)
    plsc.subcore_barrier()

    def load_idx(st, g):
        pltpu.sync_copy(e_ref.at[1, wid, g], didx[st])
        pltpu.sync_copy(e_ref.at[0, wid, g], sidx[st])

    def fire_s(st):
        for j in range(k):
            pltpu.async_copy(ones, acc_i.at[didx[st].at[j]], sems[st], add=True)
            pltpu.async_copy(ones, acc_o.at[sidx[st].at[j]], sems[st], add=True)

    def wait_s(st):
        for j in range(k):
            pltpu.make_async_copy(ones, acc_i.at[didx[st].at[j]],
                                  sems[st]).wait()
            pltpu.make_async_copy(ones, acc_o.at[sidx[st].at[j]],
                                  sems[st]).wait()

    _pipeline(steps, load_idx, lambda st: None, lambda st: None,
              fire_s, wait_s)
    plsc.subcore_barrier()
    pltpu.sync_copy(acc_i.at[pl.ds(s * _RT, _RT)],
                    oin_ref.at[c, pl.ds(s * _RT, _RT)])
    pltpu.sync_copy(acc_o.at[pl.ds(s * _RT, _RT)],
                    oout_ref.at[c, pl.ds(s * _RT, _RT)])


def _make_deg_kernel(k=_KK2):
    return pl.kernel(
        functools.partial(_deg_body, k),
        out_type=(jax.ShapeDtypeStruct((_NC, _NP), jnp.float32),
                  jax.ShapeDtypeStruct((_NC, _NP), jnp.float32)),
        mesh=_mesh(),
        compiler_params=_SC_PARAMS,
        scratch_types=(
            [pltpu.VMEM((k, _B2), jnp.int32) for _ in range(6)]
            + [pltpu.VMEM((_B2,), jnp.float32),
               pltpu.VMEM_SHARED((_NP,), jnp.float32),
               pltpu.VMEM_SHARED((_NP,), jnp.float32)]
            + [pltpu.SemaphoreType.DMA for _ in range(3)]
        ),
    )


_DI = 28
_DJ = _RT // _DI


def _agg1_body(k, e_ref, dpi_ref, dpo_ref, iidx_ref,
               z_ref, oin_ref, oout_ref, *scratch):
    sidx = scratch[0:3]
    didx = scratch[3:6]
    rin = scratch[6:9]
    rout = scratch[9:12]
    tbl_i, tbl_o, acc_i, acc_o, pbuf, iv = scratch[12:18]
    semg = scratch[18:21]
    sems = scratch[21:24]
    steps = _EW // (_B2 * k)
    c = lax.axis_index("c")
    s = lax.axis_index("s")
    wid = c * _NS + s
    sl = pl.ds(s * _RT, _RT)
    pltpu.sync_copy(dpi_ref.at[0, sl], tbl_i.at[sl])
    pltpu.sync_copy(dpo_ref.at[0, sl], tbl_o.at[sl])
    pltpu.sync_copy(iidx_ref.at[s], iv)
    _drain([pltpu.async_copy(
        dpi_ref.at[1, pl.ds(s * _RT + j * _DJ, _DJ)], pbuf.at[j], semg[0])
        for j in range(_DI)])
    for j in range(_DI):
        pltpu.sync_copy(pbuf.at[j], tbl_i.at[iv.at[j]], add=True)
    _drain([pltpu.async_copy(
        dpo_ref.at[1, pl.ds(s * _RT + j * _DJ, _DJ)], pbuf.at[j], semg[0])
        for j in range(_DI)])
    for j in range(_DI):
        pltpu.sync_copy(pbuf.at[j], tbl_o.at[iv.at[j]], add=True)
    pltpu.sync_copy(z_ref.at[sl], acc_i.at[sl])
    pltpu.sync_copy(z_ref.at[sl], acc_o.at[sl])
    plsc.subcore_barrier()

    def load_idx(st, g):
        pltpu.sync_copy(e_ref.at[0, wid, g], sidx[st])
        pltpu.sync_copy(e_ref.at[1, wid, g], didx[st])

    def fire_g(st):
        for j in range(k):
            pltpu.async_copy(tbl_i.at[sidx[st].at[j]], rin[st].at[j],
                             semg[st])
            pltpu.async_copy(tbl_o.at[sidx[st].at[j]], rout[st].at[j],
                             semg[st])

    def wait_g(st):
        for j in range(k):
            pltpu.make_async_copy(tbl_i.at[sidx[st].at[j]], rin[st].at[j],
                                  semg[st]).wait()
            pltpu.make_async_copy(tbl_o.at[sidx[st].at[j]], rout[st].at[j],
                                  semg[st]).wait()

    def fire_s(st):
        for j in range(k):
            pltpu.async_copy(rin[st].at[j], acc_i.at[didx[st].at[j]],
                             sems[st], add=True)
            pltpu.async_copy(rout[st].at[j], acc_o.at[didx[st].at[j]],
                             sems[st], add=True)

    def wait_s(st):
        for j in range(k):
            pltpu.make_async_copy(rin[st].at[j], acc_i.at[didx[st].at[j]],
                                  sems[st]).wait()
            pltpu.make_async_copy(rout[st].at[j], acc_o.at[didx[st].at[j]],
                                  sems[st]).wait()

    _pipeline(steps, load_idx, fire_g, wait_g, fire_s, wait_s)
    plsc.subcore_barrier()
    pltpu.sync_copy(acc_i.at[sl], oin_ref.at[c, sl])
    pltpu.sync_copy(acc_o.at[sl], oout_ref.at[c, sl])


def _make_agg1_kernel(k=_KK2):
    return pl.kernel(
        functools.partial(_agg1_body, k),
        out_type=(jax.ShapeDtypeStruct((_NC, _NP), jnp.float32),
                  jax.ShapeDtypeStruct((_NC, _NP), jnp.float32)),
        mesh=_mesh(),
        compiler_params=_SC_PARAMS,
        scratch_types=(
            [pltpu.VMEM((k, _B2), jnp.int32) for _ in range(6)]
            + [pltpu.VMEM((k, _B2), jnp.float32) for _ in range(6)]
            + [pltpu.VMEM_SHARED((_NP,), jnp.float32),
               pltpu.VMEM_SHARED((_NP,), jnp.float32),
               pltpu.VMEM_SHARED((_NP,), jnp.float32),
               pltpu.VMEM_SHARED((_NP,), jnp.float32),
               pltpu.VMEM((_DI, _DJ), jnp.float32),
               pltpu.VMEM((_DI, _DJ), jnp.int32)]
            + [pltpu.SemaphoreType.DMA for _ in range(6)]
        ),
    )


def _agg2_body(k, e_ref, ai_ref, ao_ref, w_ref, z_ref, out_ref, h1o_ref, *scratch):
    sidx = scratch[0:4]
    didx = scratch[4:8]
    rows = scratch[8:11]
    acc, ab, ob, wb = scratch[11:15]
    semi = scratch[15:19]
    semg = scratch[19:22]
    sems = scratch[22:25]
    steps = _EW // (_B3 * k)
    c = lax.axis_index("c")
    s = lax.axis_index("s")
    wid = c * _NS + s
    sl = pl.ds(s * _RT, _RT)
    pltpu.sync_copy(w_ref, wb)
    w0a = wb[0, pl.ds(0, 16)]
    w0b = wb[0, pl.ds(16, 16)]
    w1a = wb[1, pl.ds(0, 16)]
    w1b = wb[1, pl.ds(16, 16)]
    b1a = wb[2, pl.ds(0, 16)]
    b1b = wb[2, pl.ds(16, 16)]
    rq = _RT // 4
    for q in range(4):
        qoff = s * _RT + q * rq
        pltpu.sync_copy(ai_ref.at[0, pl.ds(qoff, rq)], ab.at[0])
        pltpu.sync_copy(ai_ref.at[1, pl.ds(qoff, rq)], ab.at[1])
        pltpu.sync_copy(ao_ref.at[0, pl.ds(qoff, rq)], ab.at[2])
        pltpu.sync_copy(ao_ref.at[1, pl.ds(qoff, rq)], ab.at[3])

        def grp(g, carry, qoff=qoff):
            base = g * 16
            ainv = ab[0, pl.ds(base, 16)] + ab[1, pl.ds(base, 16)]
            aoutv = ab[2, pl.ds(base, 16)] + ab[3, pl.ds(base, 16)]
            for n in range(16):
                ain = ainv[n]
                aout = aoutv[n]
                ob[n, pl.ds(0, 16)] = jnp.maximum(
                    ain * w0a + aout * w1a + b1a, 0.0)
                ob[n, pl.ds(16, 16)] = jnp.maximum(
                    ain * w0b + aout * w1b + b1b, 0.0)
            pltpu.sync_copy(ob, h1o_ref.at[c, pl.ds(qoff + base, 16)])
            return carry

        lax.fori_loop(0, rq // 16, grp, 0)
    pltpu.sync_copy(z_ref.at[sl], acc.at[sl])
    plsc.subcore_barrier()

    tab = h1o_ref.at[c]

    def fire_idx(it, g):
        pltpu.async_copy(e_ref.at[0, wid, g], sidx[it], semi[it])
        pltpu.async_copy(e_ref.at[1, wid, g], didx[it], semi[it])

    def wait_idx(it):
        pltpu.make_async_copy(e_ref.at[0, wid, 0], sidx[it], semi[it]).wait()
        pltpu.make_async_copy(e_ref.at[1, wid, 0], didx[it], semi[it]).wait()

    def fire_g(st, it):
        for j in range(k):
            pltpu.async_copy(tab.at[sidx[it].at[j]], rows[st].at[j],
                             semg[st])

    def wait_g(st):
        for j in range(k):
            pltpu.make_async_copy(tab.at[sidx[0].at[j]], rows[st].at[j],
                                  semg[st]).wait()

    def fire_s(st, it):
        for j in range(k):
            pltpu.async_copy(rows[st].at[j], acc.at[didx[it].at[j]],
                             sems[st], add=True)

    def wait_s(st):
        for j in range(k):
            pltpu.make_async_copy(rows[st].at[j], acc.at[didx[0].at[j]],
                                  sems[st]).wait()

    _pipeline4(steps, fire_idx, wait_idx, fire_g, wait_g, fire_s, wait_s)
    plsc.subcore_barrier()
    pltpu.sync_copy(acc.at[sl], out_ref.at[c, sl])


def _make_agg2_kernel(k=_KK3):
    return pl.kernel(
        functools.partial(_agg2_body, k),
        out_type=(jax.ShapeDtypeStruct((_NC, _NP, 32), jnp.float32),
                  jax.ShapeDtypeStruct((_NC, _NP, 32), jnp.float32)),
        mesh=_mesh(),
        compiler_params=_SC_PARAMS,
        scratch_types=(
            [pltpu.VMEM((k, _B3), jnp.int32) for _ in range(8)]
            + [pltpu.VMEM((k, _B3, 32), jnp.float32) for _ in range(3)]
            + [pltpu.VMEM_SHARED((_NP, 32), jnp.float32),
               pltpu.VMEM((4, _RT // 4), jnp.float32),
               pltpu.VMEM((16, 32), jnp.float32),
               pltpu.VMEM((3, 32), jnp.float32)]
            + [pltpu.SemaphoreType.DMA for _ in range(10)]
        ),
    )


_NB = 16


def _final_body(p_ref, w2t_ref, b2_ref, params_ref, pw1t_ref, pb1_ref,
                pw2t_ref, pb2_ref, fw1t_ref, fb1_ref, fw2t_ref, fb2_ref,
                ge_ref, pe_ref, m_ref, acc_ref):
    i = pl.program_id(0)
    agg = p_ref[0] + p_ref[1]
    h2 = jnp.dot(agg, w2t_ref[...],
                 preferred_element_type=jnp.float32) + b2_ref[...]
    r = jnp.maximum(h2, 0.0)
    row = lax.broadcasted_iota(jnp.int32, (_RT, 1), 0) + i * _RT
    r = jnp.where(row < _N, r, 0.0)
    part = jnp.sum(r, axis=0, keepdims=True)

    @pl.when(i == 0)
    def _():
        acc_ref[...] = part

    @pl.when(i > 0)
    def _():
        acc_ref[...] += part

    @pl.when(i == _NB - 1)
    def _():
        ge = acc_ref[...]
        ph = jnp.maximum(jnp.dot(params_ref[...], pw1t_ref[...],
                                 preferred_element_type=jnp.float32)
                         + pb1_ref[...], 0.0)
        pe = jnp.dot(ph, pw2t_ref[...],
                     preferred_element_type=jnp.float32) + pb2_ref[...]
        cat = jnp.concatenate([ge, pe], axis=1)
        m = jnp.maximum(jnp.dot(cat, fw1t_ref[...],
                                preferred_element_type=jnp.float32)
                        + fb1_ref[...], 0.0)
        m_ref[...] = jnp.dot(m, fw2t_ref[...],
                             preferred_element_type=jnp.float32) + fb2_ref[...]
        ge_ref[...] = ge
        pe_ref[...] = pe


def _full(shape):
    return pl.BlockSpec(shape, lambda i: tuple(0 for _ in shape))


def _make_final_call():
    f32 = jnp.float32
    return pl.pallas_call(
        _final_body,
        grid=(_NB,),
        in_specs=[
            pl.BlockSpec((_NC, _RT, 32), lambda i: (0, i, 0)),
            _full((32, 32)), _full((1, 32)), _full((1, 16)),
            _full((16, 64)), _full((1, 64)), _full((64, 32)), _full((1, 32)),
            _full((64, 64)), _full((1, 64)), _full((64, 4)), _full((1, 4)),
        ],
        out_specs=[_full((1, 32)), _full((1, 32)), _full((1, 4))],
        out_shape=[jax.ShapeDtypeStruct((1, 32), f32),
                   jax.ShapeDtypeStruct((1, 32), f32),
                   jax.ShapeDtypeStruct((1, 4), f32)],
        scratch_shapes=[pltpu.VMEM((1, 32), f32)],
    )


def kernel(edge_index, params, W1, b1, W2, b2, pW1, pb1, pW2, pb2,
           fW1, fb1, fW2, fb2):
    f32 = jnp.float32
    er2 = edge_index.reshape(2, _NW, _EW // (_B2 * _KK2), _KK2, _B2)
    er32 = edge_index.reshape(2, _NW, _EW // (_B3 * _KK3), _KK3, _B3)
    z1 = jnp.zeros((_NP,), f32)
    z32 = jnp.zeros((_NP, 32), f32)
    ones = jnp.ones((_B2,), f32)
    iidx = jnp.arange(_NP, dtype=jnp.int32).reshape(_NS, _DI, _DJ)
    wcat = jnp.stack([W1[:, 0], W1[:, 1], b1], 0)

    din_p, dout_p = _make_deg_kernel()(er2, ones, z1)
    ain_p, aout_p = _make_agg1_kernel()(er2, din_p, dout_p, iidx, z1)
    agg2_p, _h1o = _make_agg2_kernel()(er32, ain_p, aout_p, wcat, z32)
    ge, pe, metrics = _make_final_call()(
        agg2_p, W2.T, b2.reshape(1, 32), params, pW1.T, pb1.reshape(1, 64),
        pW2.T, pb2.reshape(1, 32), fW1.T, fb1.reshape(1, 64), fW2.T,
        fb2.reshape(1, 4))
    return (ge, pe, metrics)
